# Initial kernel scaffold; baseline (speedup 1.0000x reference)
import jax
import jax.numpy as jnp
from jax.experimental import pallas as pl


def kernel(x_paper, x_author, ei_pp, ei_ap, params):
    raise NotImplementedError("write your pallas kernel here")


# speedup vs baseline: 11.5975x; 11.5975x over previous
import functools
import math

import jax
import jax.numpy as jnp
from jax import lax
from jax.experimental import pallas as pl
from jax.experimental.pallas import tpu as pltpu
from jax.experimental.pallas import tpu_sc as plsc

N_NODE = 50000
N_EDGE = 300000
C_FEAT = 128
HEADS = 4
D_HEAD = 32

NC, NS = 2, 16
NW = NC * NS
K_CH = 128
E_PAD = ((N_EDGE + NW * K_CH - 1) // (NW * K_CH)) * (NW * K_CH)
CH_PER_W = E_PAD // (NW * K_CH)
BM = 512
N_PAD = ((N_NODE + BM - 1) // BM) * BM
DUMMY = N_NODE + 8

def _sc_mesh():
    return plsc.VectorSubcoreMesh(core_axis_name="c", subcore_axis_name="s",
                                  num_cores=NC, num_subcores=NS)


def _mm_bias_body(x_ref, w_ref, b_ref, o_ref):
    o_ref[...] = (
        jnp.dot(x_ref[...], w_ref[...], preferred_element_type=jnp.float32,
                precision=lax.Precision.HIGHEST)
        + b_ref[...]
    )


def _tc_mm_bias(x, w, b):
    rows = x.shape[0]
    return pl.pallas_call(
        _mm_bias_body,
        grid=(rows // BM,),
        in_specs=[
            pl.BlockSpec((BM, C_FEAT), lambda i: (i, 0)),
            pl.BlockSpec((C_FEAT, C_FEAT), lambda i: (0, 0)),
            pl.BlockSpec((1, C_FEAT), lambda i: (0, 0)),
        ],
        out_specs=pl.BlockSpec((BM, C_FEAT), lambda i: (i, 0)),
        out_shape=jax.ShapeDtypeStruct((rows, C_FEAT), jnp.float32),
    )(x, w, b.reshape(1, C_FEAT))


def _out_proj_body(u_ref, x2_ref, w_ref, b_ref, o_ref):
    o_ref[...] = (
        jnp.dot(jax.nn.gelu(u_ref[...]), w_ref[...],
                preferred_element_type=jnp.float32,
                precision=lax.Precision.HIGHEST)
        + b_ref[...]
        + x2_ref[...]
    )


def _tc_out_proj(u, x2v, w, b):
    rows = u.shape[0]
    return pl.pallas_call(
        _out_proj_body,
        grid=(rows // BM,),
        in_specs=[
            pl.BlockSpec((BM, C_FEAT), lambda i: (i, 0)),
            pl.BlockSpec((BM, C_FEAT), lambda i: (i, 0)),
            pl.BlockSpec((C_FEAT, C_FEAT), lambda i: (0, 0)),
            pl.BlockSpec((1, C_FEAT), lambda i: (0, 0)),
        ],
        out_specs=pl.BlockSpec((BM, C_FEAT), lambda i: (i, 0)),
        out_shape=jax.ShapeDtypeStruct((rows, C_FEAT), jnp.float32),
    )(u, x2v, w, b.reshape(1, C_FEAT))


def _logits_body(q_ref, k_ref, s_ref, o_ref):
    o_ref[...] = jnp.dot(q_ref[...] * k_ref[...], s_ref[...],
                         preferred_element_type=jnp.float32,
                         precision=lax.Precision.HIGHEST)


def _tc_logits(qd, ks, sel):
    rows = qd.shape[0]
    return pl.pallas_call(
        _logits_body,
        grid=(rows // BM,),
        in_specs=[
            pl.BlockSpec((BM, C_FEAT), lambda i: (i, 0)),
            pl.BlockSpec((BM, C_FEAT), lambda i: (i, 0)),
            pl.BlockSpec((C_FEAT, 8), lambda i: (0, 0)),
        ],
        out_specs=pl.BlockSpec((BM, 8), lambda i: (i, 0)),
        out_shape=jax.ShapeDtypeStruct((rows, 8), jnp.float32),
    )(qd, ks, sel)


def _gather_body(table, idx, out, idx_v, rows_v, sem):
    wid = lax.axis_index("s") * NC + lax.axis_index("c")

    def chunk(i, c):
        off = (wid * CH_PER_W + i) * K_CH
        pltpu.sync_copy(idx.at[pl.ds(off, K_CH)], idx_v)
        pltpu.async_copy(table.at[idx_v], rows_v, sem).wait()
        pltpu.sync_copy(rows_v, out.at[pl.ds(off, K_CH)])
        return c

    lax.fori_loop(0, CH_PER_W, chunk, 0)


def _sc_gather(table, idx, ncols):
    f = pl.kernel(
        _gather_body,
        out_type=jax.ShapeDtypeStruct((E_PAD, ncols), jnp.float32),
        mesh=_sc_mesh(),
        scratch_types=[
            pltpu.VMEM((K_CH,), jnp.int32),
            pltpu.VMEM((K_CH, ncols), jnp.float32),
            pltpu.SemaphoreType.DMA,
        ],
    )
    return f(table, idx)


def _scatter_body(vals, idx, zeros, out, idx_v, vals_v, shared, *, nrows):
    cid = lax.axis_index("c")
    sid = lax.axis_index("s")
    wid = sid * NC + cid
    rps = nrows // NS
    pltpu.sync_copy(zeros.at[pl.ds(sid * rps, rps)],
                    shared.at[pl.ds(sid * rps, rps)])
    plsc.subcore_barrier()

    def chunk(i, c):
        off = (wid * CH_PER_W + i) * K_CH
        pltpu.sync_copy(idx.at[pl.ds(off, K_CH)], idx_v)
        pltpu.sync_copy(vals.at[pl.ds(off, K_CH)], vals_v)
        pltpu.sync_copy(vals_v, shared.at[idx_v], add=True)
        return c

    lax.fori_loop(0, CH_PER_W, chunk, 0)
    plsc.subcore_barrier()
    pltpu.sync_copy(shared.at[pl.ds(sid * rps, rps)],
                    out.at[cid, pl.ds(sid * rps, rps)])


def _sc_scatter_add(vals, idx, nrows):
    zeros = jnp.zeros((nrows, 128), jnp.float32)
    f = pl.kernel(
        functools.partial(_scatter_body, nrows=nrows),
        out_type=jax.ShapeDtypeStruct((NC, nrows, 128), jnp.float32),
        mesh=_sc_mesh(),
        scratch_types=[
            pltpu.VMEM((K_CH,), jnp.int32),
            pltpu.VMEM((K_CH, 128), jnp.float32),
            pltpu.VMEM_SHARED((nrows, 128), jnp.float32),
        ],
    )
    return f(vals, idx, zeros)


def _blockdiag(rel):
    return jax.scipy.linalg.block_diag(*[rel[h] for h in range(HEADS)])


def _selector(p_rel):
    base = jnp.kron(jnp.eye(HEADS, dtype=jnp.float32),
                    jnp.ones((D_HEAD, 1), jnp.float32))
    s4 = base * (p_rel / math.sqrt(D_HEAD))[None, :]
    return jnp.pad(s4, ((0, 0), (0, 4)))


def _pad_rows(x, rows):
    return jnp.pad(x, ((0, rows - x.shape[0]), (0, 0)))


def _pad_idx(idx, fill):
    return jnp.pad(idx, (0, E_PAD - idx.shape[0]), constant_values=fill)


_ET = (("paper", "paper", "ei_pp"), ("author", "paper", "ei_ap"))
_NT = ("paper", "author")


def _hgt_layer(x, ei, p):
    q = {}
    for t in _NT:
        q[t] = _tc_mm_bias(x[t], p["q_W"][t], p["q_b"][t])
    agg = {t: jnp.zeros((N_PAD, C_FEAT), jnp.float32) for t in _NT}
    for (s, d, en_key) in _ET:
        en = {"ei_pp": "paper__cites__paper",
              "ei_ap": "author__writes__paper"}[en_key]
        a_bd = _blockdiag(p["a_rel"][en])
        m_bd = _blockdiag(p["m_rel"][en])
        kt_t = _tc_mm_bias(x[s], p["k_W"][s] @ a_bd, p["k_b"][s] @ a_bd)
        vt_t = _tc_mm_bias(x[s], p["v_W"][s] @ m_bd, p["v_b"][s] @ m_bd)
        e = ei[en_key]
        src = _pad_idx(e[0], 0)
        dst_g = _pad_idx(e[1], 0)
        dst_s = _pad_idx(e[1], DUMMY)
        ks_e = _sc_gather(kt_t, src, C_FEAT)
        qd_e = _sc_gather(q[d], dst_g, C_FEAT)
        logits = _tc_logits(qd_e, ks_e, _selector(p["p_rel"][en]))[:, :HEADS]
        gmax = jnp.max(logits[:N_EDGE])
        ex = jnp.exp(logits - gmax)
        ex16 = jnp.pad(ex, ((0, 0), (0, 12)))
        oh8 = (dst_s[:, None] % 8 == jnp.arange(8)[None, :]).astype(jnp.float32)
        ex128 = (oh8[:, :, None] * ex16[:, None, :]).reshape(E_PAD, 128)
        dp = _sc_scatter_add(ex128, dst_s // 8, N_PAD // 8)
        den_et = (dp[0] + dp[1]).reshape(N_PAD, 16)[:, :HEADS]
        vs_e = _sc_gather(vt_t, src, C_FEAT)
        contrib = vs_e * jnp.repeat(ex, D_HEAD, axis=1)
        oh4 = (dst_s[:, None] % 4 == jnp.arange(4)[None, :]).astype(jnp.float32)
        inv = jnp.repeat(1.0 / (den_et + 1e-16), D_HEAD, axis=1)
        for c in range(HEADS):
            c128 = (oh4[:, :, None]
                    * contrib[:, None, c * D_HEAD:(c + 1) * D_HEAD]
                    ).reshape(E_PAD, 128)
            mp = _sc_scatter_add(c128, dst_s // 4, N_PAD // 4)
            agg[d] = agg[d].at[:, c * D_HEAD:(c + 1) * D_HEAD].add(
                (mp[0] + mp[1]).reshape(N_PAD, 32)
                * inv[:, c * D_HEAD:(c + 1) * D_HEAD])
    res = {}
    for t in _NT:
        a = jax.nn.sigmoid(p["skip"][t])
        res[t] = _tc_out_proj(agg[t], (1.0 - a) * x[t],
                              a * p["a_W"][t], a * p["a_b"][t])
    return res


def kernel(x_paper, x_author, ei_pp, ei_ap, params):
    x = {
        "paper": _tc_mm_bias(_pad_rows(x_paper, N_PAD),
                             params["lin"]["paper"][0],
                             params["lin"]["paper"][1]),
        "author": _tc_mm_bias(_pad_rows(x_author, N_PAD),
                              params["lin"]["author"][0],
                              params["lin"]["author"][1]),
    }
    ei = {"ei_pp": ei_pp, "ei_ap": ei_ap}
    x = _hgt_layer(x, ei, params["conv1"])
    x = _hgt_layer(x, ei, params["conv2"])
    return (x["paper"][:N_NODE], x["author"][:N_NODE])
